# Initial kernel scaffold; baseline (speedup 1.0000x reference)
import jax
import jax.numpy as jnp
from jax.experimental import pallas as pl


def kernel(xs, embed_weight):
    raise NotImplementedError("write your pallas kernel here")


# speedup vs baseline: 1.0248x; 1.0248x over previous
import functools

import jax
import jax.numpy as jnp
from jax import lax
from jax.experimental import pallas as pl
from jax.experimental.pallas import tpu as pltpu
from jax.experimental.pallas import tpu_sc as plsc

_NVEC1 = 1000
_IDX_ROW = 128


def kernel(xs, embed_weight):
    B = xs.shape[0]
    V = embed_weight.shape[0]
    info = plsc.get_sparse_core_info()
    NC, NS, L = info.num_cores, info.num_subcores, info.num_lanes
    NW = NC * NS
    bpw = B // NW
    ngrp = bpw // L
    nrow = bpw // _IDX_ROW
    gprow = _IDX_ROW // L

    mesh = plsc.VectorSubcoreMesh(core_axis_name="c", subcore_axis_name="s")

    @functools.partial(
        pl.kernel,
        mesh=mesh,
        out_type=jax.ShapeDtypeStruct((B,), jnp.float32),
        scratch_types=[
            pltpu.VMEM((bpw,), jnp.int32),
            pltpu.VMEM((bpw,), jnp.int32),
            pltpu.VMEM((nrow, _IDX_ROW), jnp.int32),
            pltpu.VMEM((bpw,), jnp.float32),
            pltpu.SemaphoreType.DMA,
        ],
    )
    def _gather(x0_hbm, x1_hbm, tbl_hbm, out_hbm, x0_v, x1_v, idx_v, vals_v,
                sem):
        wid = lax.axis_index("s") * NC + lax.axis_index("c")
        base = wid * bpw
        pltpu.sync_copy(x0_hbm.at[pl.ds(base, bpw)], x0_v)
        pltpu.sync_copy(x1_hbm.at[pl.ds(base, bpw)], x1_v)
        for i in range(ngrp):
            s = pl.ds(i * L, L)
            idx_v[i // gprow, pl.ds((i % gprow) * L, L)] = (
                x0_v[s] * _NVEC1 + x1_v[s]
            )
        copies = [
            pltpu.async_copy(
                tbl_hbm.at[idx_v.at[j]],
                vals_v.at[pl.ds(j * _IDX_ROW, _IDX_ROW)],
                sem,
            )
            for j in range(nrow)
        ]
        for c in copies:
            c.wait()
        pltpu.sync_copy(vals_v, out_hbm.at[pl.ds(base, bpw)])

    return _gather(xs[:, 0], xs[:, 1], embed_weight.reshape(V))


# speedup vs baseline: 1.0623x; 1.0366x over previous
import functools

import jax
import jax.numpy as jnp
from jax import lax
from jax.experimental import pallas as pl
from jax.experimental.pallas import tpu as pltpu
from jax.experimental.pallas import tpu_sc as plsc

_NVEC1 = 1000
_CHUNK = 128


def kernel(xs, embed_weight):
    B = xs.shape[0]
    info = plsc.get_sparse_core_info()
    NC, NS, L = info.num_cores, info.num_subcores, info.num_lanes
    NW = NC * NS
    bpw = B // NW
    nchunk = bpw // _CHUNK
    gpc = _CHUNK // L

    mesh = plsc.VectorSubcoreMesh(core_axis_name="c", subcore_axis_name="s")

    @functools.partial(
        pl.kernel,
        mesh=mesh,
        out_type=jax.ShapeDtypeStruct((B,), jnp.float32),
        scratch_types=[
            pltpu.VMEM((2 * bpw,), jnp.int32),
            pltpu.VMEM((nchunk, _CHUNK), jnp.int32),
            pltpu.VMEM((bpw,), jnp.float32),
            pltpu.SemaphoreType.DMA,
            pltpu.SemaphoreType.DMA,
        ],
    )
    def _gather(xsf_hbm, tbl_hbm, out_hbm, xs_v, idx_v, vals_v, gsem, osem):
        wid = lax.axis_index("s") * NC + lax.axis_index("c")
        base = wid * bpw
        pltpu.sync_copy(xsf_hbm.at[pl.ds(2 * base, 2 * bpw)], xs_v)
        gathers = []
        for k in range(nchunk):
            for g in range(gpc):
                v0 = xs_v[pl.ds(2 * _CHUNK * k + L * g, L)]
                v1 = xs_v[pl.ds(2 * _CHUNK * k + _CHUNK + L * g, L)]
                idx_v[k, pl.ds(L * g, L)] = v0 * _NVEC1 + v1
            gathers.append(
                pltpu.async_copy(
                    tbl_hbm.at[idx_v.at[k]],
                    vals_v.at[pl.ds(k * _CHUNK, _CHUNK)],
                    gsem,
                )
            )
        writes = []
        for k in range(nchunk):
            gathers[k].wait()
            writes.append(
                pltpu.async_copy(
                    vals_v.at[pl.ds(k * _CHUNK, _CHUNK)],
                    out_hbm.at[pl.ds(base + k * _CHUNK, _CHUNK)],
                    osem,
                )
            )
        for w in writes:
            w.wait()

    xsf = xs.reshape(B // _CHUNK, _CHUNK, 2).transpose(0, 2, 1).reshape(2 * B)
    return _gather(xsf, embed_weight.reshape(-1))
